# baseline (speedup 1.0000x reference)
import functools

import jax
import jax.numpy as jnp
from jax import lax
from jax.experimental import pallas as pl
from jax.experimental.pallas import tpu as pltpu
from jax.experimental.pallas import tpu_sc as plsc

DIM = 32
BATCH_ = 16384
HIST = 50

NC = 2
NS = 16
NW = NC * NS
B_PER_W = BATCH_ // NW
CHUNK = 32
ROWS = CHUNK * HIST
ROUNDS = B_PER_W // CHUNK


def _sc_kernel(table_hbm, idx_hbm, out_hbm,
               idx2d0, idx2d1, idxf0, idxf1, rows0, rows1, out_v,
               sem0, sem1):
    wid = lax.axis_index("s") * NC + lax.axis_index("c")
    base_b0 = wid * B_PER_W
    zero = jnp.zeros((16,), jnp.float32)
    inv = jnp.float32(1.0 / HIST)
    idx2d_b = (idx2d0, idx2d1)
    idxf_b = (idxf0, idxf1)
    rows_b = (rows0, rows1)
    sems = (sem0, sem1)

    def start(r, p):
        idx2d = idx2d_b[p]
        idxf = idxf_b[p]
        pltpu.sync_copy(idx_hbm.at[pl.ds(base_b0 + r * CHUNK, CHUNK), :],
                        idx2d)

        def flat_body(b, c):
            base = b * HIST
            idxf[pl.ds(base, 16)] = idx2d[b, pl.ds(0, 16)]
            idxf[pl.ds(base + 16, 16)] = idx2d[b, pl.ds(16, 16)]
            idxf[pl.ds(base + 32, 16)] = idx2d[b, pl.ds(32, 16)]
            idxf[pl.ds(base + HIST - 16, 16)] = idx2d[b, pl.ds(HIST - 16, 16)]
            return c

        lax.fori_loop(0, CHUNK, flat_body, 0)
        pltpu.async_copy(table_hbm.at[idxf], rows_b[p], sems[p])

    def process(r, p):
        rows_v = rows_b[p]

        def batch_body(b, c):
            base_row = b * HIST
            a0 = zero
            a1 = zero
            c0 = zero
            c1 = zero
            for h in range(0, HIST, 2):
                a0 = a0 + rows_v[base_row + h, pl.ds(0, 16)]
                a1 = a1 + rows_v[base_row + h, pl.ds(16, 16)]
                c0 = c0 + rows_v[base_row + h + 1, pl.ds(0, 16)]
                c1 = c1 + rows_v[base_row + h + 1, pl.ds(16, 16)]
            out_v[b, pl.ds(0, 16)] = (a0 + c0) * inv
            out_v[b, pl.ds(16, 16)] = (a1 + c1) * inv
            return c

        lax.fori_loop(0, CHUNK, batch_body, 0)
        pltpu.sync_copy(out_v, out_hbm.at[pl.ds(base_b0 + r * CHUNK, CHUNK)])

    start(0, 0)

    def outer(rr, carry):
        for p in (0, 1):
            r = rr * 2 + p
            nxt = (p + 1) % 2

            @pl.when(r + 1 < ROUNDS)
            def _():
                start(r + 1, nxt)

            pltpu.make_async_copy(table_hbm.at[idxf_b[p]], rows_b[p],
                                  sems[p]).wait()
            process(r, p)
        return carry

    lax.fori_loop(0, ROUNDS // 2, outer, 0)


@jax.jit
def _pooled_lookup(emb_table, idx2d):
    mesh = plsc.VectorSubcoreMesh(core_axis_name="c", subcore_axis_name="s")
    f = functools.partial(
        pl.kernel,
        mesh=mesh,
        out_type=jax.ShapeDtypeStruct((BATCH_, DIM), jnp.float32),
        scratch_types=[
            pltpu.VMEM((CHUNK, HIST), jnp.int32),
            pltpu.VMEM((CHUNK, HIST), jnp.int32),
            pltpu.VMEM((ROWS,), jnp.int32),
            pltpu.VMEM((ROWS,), jnp.int32),
            pltpu.VMEM((ROWS, DIM), jnp.float32),
            pltpu.VMEM((ROWS, DIM), jnp.float32),
            pltpu.VMEM((CHUNK, DIM), jnp.float32),
            pltpu.SemaphoreType.DMA,
            pltpu.SemaphoreType.DMA,
        ],
        compiler_params=pltpu.CompilerParams(use_tc_tiling_on_sc=False),
    )(_sc_kernel)
    return f(emb_table, idx2d)


def kernel(emb_table, inputs):
    return _pooled_lookup(emb_table, inputs)


# speedup vs baseline: 1.0163x; 1.0163x over previous
import functools

import jax
import jax.numpy as jnp
from jax import lax
from jax.experimental import pallas as pl
from jax.experimental.pallas import tpu as pltpu
from jax.experimental.pallas import tpu_sc as plsc

DIM = 32
BATCH_ = 16384
HIST = 50

NC = 2
NS = 16
NW = NC * NS
B_PER_W = BATCH_ // NW
CHUNK = 32
ROWS = CHUNK * HIST
ROUNDS = B_PER_W // CHUNK


def _sc_kernel(table_hbm, idxt_hbm, outt_hbm,
               idx2d0, idx2d1, idxf0, idxf1, rows0, rows1, outt_v,
               sem0, sem1):
    wid = lax.axis_index("s") * NC + lax.axis_index("c")
    base_b0 = wid * B_PER_W
    zero = jnp.zeros((16,), jnp.float32)
    inv = jnp.float32(1.0 / HIST)
    iota16 = lax.iota(jnp.int32, 16)
    idx2d_b = (idx2d0, idx2d1)
    idxf_b = (idxf0, idxf1)
    rows_b = (rows0, rows1)
    sems = (sem0, sem1)

    def start(r, p):
        idx2d = idx2d_b[p]
        idxf = idxf_b[p]
        pltpu.sync_copy(idxt_hbm.at[:, pl.ds(base_b0 + r * CHUNK, CHUNK)],
                        idx2d)

        def flat_body(h, c):
            base = h * CHUNK
            idxf[pl.ds(base, 16)] = idx2d[h, pl.ds(0, 16)]
            idxf[pl.ds(base + 16, 16)] = idx2d[h, pl.ds(16, 16)]
            return c

        lax.fori_loop(0, HIST, flat_body, 0)
        pltpu.async_copy(table_hbm.at[idxf], rows_b[p], sems[p])

    def process(r, p):
        rows_v = rows_b[p]

        def batch_body(b, c):
            a0 = zero
            a1 = zero
            c0 = zero
            c1 = zero
            for h in range(0, HIST, 2):
                a0 = a0 + rows_v[h * CHUNK + b, pl.ds(0, 16)]
                a1 = a1 + rows_v[h * CHUNK + b, pl.ds(16, 16)]
                c0 = c0 + rows_v[(h + 1) * CHUNK + b, pl.ds(0, 16)]
                c1 = c1 + rows_v[(h + 1) * CHUNK + b, pl.ds(16, 16)]
            bvec = jnp.full((16,), b, jnp.int32)
            plsc.store_scatter(outt_v, [iota16, bvec], (a0 + c0) * inv)
            plsc.store_scatter(outt_v, [iota16 + 16, bvec], (a1 + c1) * inv)
            return c

        lax.fori_loop(0, CHUNK, batch_body, 0)
        pltpu.sync_copy(outt_v,
                        outt_hbm.at[:, pl.ds(base_b0 + r * CHUNK, CHUNK)])

    start(0, 0)

    def outer(rr, carry):
        for p in (0, 1):
            r = rr * 2 + p
            nxt = (p + 1) % 2

            @pl.when(r + 1 < ROUNDS)
            def _():
                start(r + 1, nxt)

            pltpu.make_async_copy(table_hbm.at[idxf_b[p]], rows_b[p],
                                  sems[p]).wait()
            process(r, p)
        return carry

    lax.fori_loop(0, ROUNDS // 2, outer, 0)


@jax.jit
def _pooled_lookup(emb_table, idx_t):
    mesh = plsc.VectorSubcoreMesh(core_axis_name="c", subcore_axis_name="s")
    f = functools.partial(
        pl.kernel,
        mesh=mesh,
        out_type=jax.ShapeDtypeStruct((DIM, BATCH_), jnp.float32),
        scratch_types=[
            pltpu.VMEM((HIST, CHUNK), jnp.int32),
            pltpu.VMEM((HIST, CHUNK), jnp.int32),
            pltpu.VMEM((ROWS,), jnp.int32),
            pltpu.VMEM((ROWS,), jnp.int32),
            pltpu.VMEM((ROWS, DIM), jnp.float32),
            pltpu.VMEM((ROWS, DIM), jnp.float32),
            pltpu.VMEM((DIM, CHUNK), jnp.float32),
            pltpu.SemaphoreType.DMA,
            pltpu.SemaphoreType.DMA,
        ],
        compiler_params=pltpu.CompilerParams(use_tc_tiling_on_sc=False, needs_layout_passes=False),
    )(_sc_kernel)
    return f(emb_table, idx_t)


def kernel(emb_table, inputs):
    out_t = _pooled_lookup(emb_table, inputs.T)
    return out_t.T
